# Initial kernel scaffold; baseline (speedup 1.0000x reference)
import jax
import jax.numpy as jnp
from jax.experimental import pallas as pl


def kernel(mg_embedding, var_indices, patch_idx):
    raise NotImplementedError("write your pallas kernel here")


# speedup vs baseline: 13.6122x; 13.6122x over previous
import jax
import jax.numpy as jnp
from jax import lax
from jax.experimental import pallas as pl
from jax.experimental.pallas import tpu as pltpu
from jax.experimental.pallas import tpu_sc as plsc

N_VAR = 4
N_POINTS = 49152
D = 128
B = 2
V = 2
P = 4096

NC = 2
NS = 16
NW = NC * NS
ROWS_PER_W = (B * V * P) // NW
CH = 128
NCH = ROWS_PER_W // CH


def _gather_body(table_hbm, var_hbm, patch_hbm, out_hbm,
                 idx0, idx1, idx2, idx3, rows0, rows1, rows2, rows3,
                 var_v, sem):
    idxs = (idx0, idx1, idx2, idx3)
    rows = (rows0, rows1, rows2, rows3)
    c = lax.axis_index("c")
    s = lax.axis_index("s")
    w = s * NC + c
    b = (w // 8) // V
    chunk = w % 8

    for j in range(NCH):
        pltpu.sync_copy(patch_hbm.at[b, chunk, j], idxs[j])
    pltpu.sync_copy(var_hbm.at[w], var_v)

    off = var_v[...] * N_POINTS

    for j in range(NCH):
        for i in range(CH // 16):
            sl = pl.ds(i * 16, 16)
            idxs[j][sl] = idxs[j][sl] + off

    copies = [
        pltpu.async_copy(table_hbm.at[idxs[j]], rows[j], sem)
        for j in range(NCH)
    ]
    for cp in copies:
        cp.wait()

    for j in range(NCH):
        pltpu.sync_copy(rows[j], out_hbm.at[w, j])


def kernel(mg_embedding, var_indices, patch_idx):
    table2d = mg_embedding.reshape(N_VAR * N_POINTS, D)
    var_tab = jnp.broadcast_to(
        jnp.repeat(var_indices.reshape(-1).astype(jnp.int32), NW // (B * V))[:, None],
        (NW, 16),
    )
    patch4 = patch_idx.astype(jnp.int32).reshape(B, P // ROWS_PER_W, NCH, CH)

    run = pl.kernel(
        _gather_body,
        out_type=jax.ShapeDtypeStruct((NW, NCH, CH, D), jnp.float32),
        mesh=plsc.VectorSubcoreMesh(core_axis_name="c", subcore_axis_name="s"),
        scratch_types=(
            [pltpu.VMEM((CH,), jnp.int32) for _ in range(NCH)]
            + [pltpu.VMEM((CH, D), jnp.float32) for _ in range(NCH)]
            + [pltpu.VMEM((16,), jnp.int32), pltpu.SemaphoreType.DMA]
        ),
    )
    out = run(table2d, var_tab, patch4)
    return out.reshape(B, V, P, D)[:, :, None, :, :]
